# baseline (speedup 1.0000x reference)
import jax
import jax.numpy as jnp
from jax.experimental import pallas as pl
from jax.experimental.pallas import tpu as pltpu

_N = 4096
_C = 151
_H = 64
_TAKE = 64
_NBLK = 8
_BW = _N // _NBLK
_NEG = float("-inf")
_POS = float("inf")
_IBIG = 1 << 30


def _mlp(x, W1, b1, W2, b2):
    h = jnp.maximum(jax.lax.dot_general(
        x, W1, (((1,), (0,)), ((), ())),
        preferred_element_type=jnp.float32) + b1, 0.0)
    return jax.lax.dot_general(
        h, W2, (((1,), (0,)), ((), ())),
        preferred_element_type=jnp.float32) + b2


def _relpn_kernel(x_ref, W1s_ref, b1s_ref, W2s_ref, b2s_ref,
                  W1o_ref, b1o_ref, W2o_ref, b2o_ref,
                  pairs_ref, scores_ref, L_ref):
    x = x_ref[0]
    subj = _mlp(x, W1s_ref[...], b1s_ref[...], W2s_ref[...], b2s_ref[...])
    obj = _mlp(x, W1o_ref[...], b1o_ref[...], W2o_ref[...], b2o_ref[...])

    parts = []
    for blk in range(_NBLK):
        sblk = subj[blk * _BW:(blk + 1) * _BW, :]
        lbT = jax.lax.dot_general(
            obj, sblk, (((1,), (1,)), ((), ())),
            preferred_element_type=jnp.float32)
        parts.append(jnp.max(lbT, axis=0, keepdims=True))
    rowmax = jnp.concatenate(parts, axis=0)

    rowid = (jax.lax.broadcasted_iota(jnp.int32, (_NBLK, _BW), 0) * _BW
             + jax.lax.broadcasted_iota(jnp.int32, (_NBLK, _BW), 1))
    lane64 = jax.lax.broadcasted_iota(jnp.int32, (1, _TAKE), 1)
    sub64 = jax.lax.broadcasted_iota(jnp.int32, (_TAKE, 1), 0)
    col4096 = jax.lax.broadcasted_iota(jnp.int32, (1, _N), 1)

    bits = jax.lax.bitcast_convert_type(jax.nn.sigmoid(rowmax), jnp.int32)
    msb = jnp.int32(-(2 ** 31))
    srm = jnp.where(bits >= 0, bits, jnp.bitwise_xor(~bits, msb))
    t_u = jnp.zeros((), jnp.int32)
    for b in range(31, -1, -1):
        t_try = jnp.bitwise_or(t_u, jnp.int32(-(2 ** 31) if b == 31 else 1 << b))
        thr = jnp.bitwise_xor(t_try, msb)
        cnt = jnp.sum((srm >= thr).astype(jnp.int32))
        t_u = jnp.where(cnt >= _TAKE, t_try, t_u)
    t_s = jnp.bitwise_xor(t_u, msb)

    gt = srm > t_s
    eq = srm == t_s
    n_gt = jnp.sum(gt.astype(jnp.int32))
    need = _TAKE - n_gt

    li = jax.lax.broadcasted_iota(jnp.int32, (_BW, _BW), 0)
    lj = jax.lax.broadcasted_iota(jnp.int32, (_BW, _BW), 1)
    ut = (li <= lj).astype(jnp.float32)
    si = jax.lax.broadcasted_iota(jnp.int32, (_NBLK, _NBLK), 0)
    sj = jax.lax.broadcasted_iota(jnp.int32, (_NBLK, _NBLK), 1)
    lt = (sj < si).astype(jnp.float32)

    def excl_prefix(mask):
        mf = mask.astype(jnp.float32)
        incl = jax.lax.dot_general(mf, ut, (((1,), (0,)), ((), ())),
                                   preferred_element_type=jnp.float32)
        totals = jnp.sum(mf, axis=1, keepdims=True)
        offs = jax.lax.dot_general(lt, totals, (((1,), (0,)), ((), ())),
                                   preferred_element_type=jnp.float32)
        return (incl - mf + offs).astype(jnp.int32)

    eq_take = eq & (excl_prefix(eq) < need)
    sel = gt | eq_take
    slot = excl_prefix(sel)

    gath = jnp.zeros((_TAKE, _H), jnp.float32)
    ids = jnp.zeros((_TAKE, 1), jnp.float32)
    for blk in range(_NBLK):
        pb = (jnp.where(sel[blk:blk + 1, :], slot[blk:blk + 1, :], _IBIG)
              == sub64).astype(jnp.float32)
        gath = gath + jax.lax.dot_general(
            pb, subj[blk * _BW:(blk + 1) * _BW, :], (((1,), (0,)), ((), ())),
            preferred_element_type=jnp.float32)
        idc = jax.lax.broadcasted_iota(jnp.int32, (_BW, 1), 0) + blk * _BW
        idlo = (idc % 256).astype(jnp.float32)
        idhi = (idc // 256).astype(jnp.float32)
        ids = (ids
               + jax.lax.dot_general(pb, idlo, (((1,), (0,)), ((), ())),
                                     preferred_element_type=jnp.float32)
               + 256.0 * jax.lax.dot_general(pb, idhi, (((1,), (0,)), ((), ())),
                                             preferred_element_type=jnp.float32))
    sel_col = ids.astype(jnp.int32)

    L = jax.lax.dot_general(gath, obj, (((1,), (1,)), ((), ())),
                            preferred_element_type=jnp.float32)
    S = jax.nn.sigmoid(L)
    L_ref[...] = S
    lrm = jnp.max(S, axis=1, keepdims=True)

    def pop(t, carry):
        lrm, scores, rvec, cvec = carry
        m = jnp.max(lrm)
        ro = jnp.min(jnp.where(lrm == m, sel_col, _IBIG))
        tloc = jnp.min(jnp.where((lrm == m) & (sel_col == ro), sub64, _IBIG))
        lrow = L_ref[pl.ds(tloc, 1), :]
        c = jnp.min(jnp.where(lrow == m, col4096, _IBIG))
        lrow2 = jnp.where(col4096 == c, _NEG, lrow)
        L_ref[pl.ds(tloc, 1), :] = lrow2
        lrm = jnp.where(sub64 == tloc, jnp.max(lrow2), lrm)
        scores = jnp.where(lane64 == t, m, scores)
        rvec = jnp.where(lane64 == t, ro, rvec)
        cvec = jnp.where(lane64 == t, c, cvec)
        return lrm, scores, rvec, cvec

    _, scores, rvec, cvec = jax.lax.fori_loop(
        0, _TAKE, pop,
        (lrm,
         jnp.full((1, _TAKE), _NEG, jnp.float32),
         jnp.zeros((1, _TAKE), jnp.int32),
         jnp.zeros((1, _TAKE), jnp.int32)))

    pairs_ref[0, 0:1, :] = rvec
    pairs_ref[0, 1:2, :] = cvec
    scores_ref[0, 0:1, :] = scores


@jax.jit
def kernel(class_logits, proposals, W1s, b1s, W2s, b2s, W1o, b1o, W2o, b2o):
    del proposals
    B = class_logits.shape[0]
    b1s2 = b1s.reshape(1, _H)
    b2s2 = b2s.reshape(1, _H)
    b1o2 = b1o.reshape(1, _H)
    b2o2 = b2o.reshape(1, _H)

    full = lambda shape: pl.BlockSpec(shape, lambda b: (0,) * len(shape))
    pairs2, scores = pl.pallas_call(
        _relpn_kernel,
        grid=(B,),
        in_specs=[
            pl.BlockSpec((1, _N, _C), lambda b: (b, 0, 0)),
            full((_C, _H)), full((1, _H)), full((_H, _H)), full((1, _H)),
            full((_C, _H)), full((1, _H)), full((_H, _H)), full((1, _H)),
        ],
        out_specs=[
            pl.BlockSpec((1, 2, _TAKE), lambda b: (b, 0, 0)),
            pl.BlockSpec((1, 1, _TAKE), lambda b: (b, 0, 0)),
        ],
        out_shape=[
            jax.ShapeDtypeStruct((B, 2, _TAKE), jnp.int32),
            jax.ShapeDtypeStruct((B, 1, _TAKE), jnp.float32),
        ],
        scratch_shapes=[
            pltpu.VMEM((_TAKE, _N), jnp.float32),
        ],
        compiler_params=pltpu.CompilerParams(
            dimension_semantics=("arbitrary",),
        ),
    )(class_logits, W1s, b1s2, W2s, b2s2, W1o, b1o2, W2o, b2o2)

    pairs = jnp.swapaxes(pairs2, 1, 2)
    return pairs, scores[:, 0, :]


# speedup vs baseline: 1.0109x; 1.0109x over previous
import jax
import jax.numpy as jnp
from jax.experimental import pallas as pl
from jax.experimental.pallas import tpu as pltpu

_N = 4096
_C = 151
_H = 64
_TAKE = 64
_NBLK = 8
_BW = _N // _NBLK
_NEG = float("-inf")
_POS = float("inf")
_IBIG = 1 << 30


def _mlp(x, W1, b1, W2, b2):
    h = jnp.maximum(jax.lax.dot_general(
        x, W1, (((1,), (0,)), ((), ())),
        preferred_element_type=jnp.float32) + b1, 0.0)
    return jax.lax.dot_general(
        h, W2, (((1,), (0,)), ((), ())),
        preferred_element_type=jnp.float32) + b2


def _prep_image(x, W1s, b1s, W2s, b2s, W1o, b1o, W2o, b2o):
    subj = _mlp(x, W1s, b1s, W2s, b2s)
    obj = _mlp(x, W1o, b1o, W2o, b2o)

    parts = []
    for blk in range(_NBLK):
        sblk = subj[blk * _BW:(blk + 1) * _BW, :]
        lbT = jax.lax.dot_general(
            obj, sblk, (((1,), (1,)), ((), ())),
            preferred_element_type=jnp.float32)
        parts.append(jnp.max(lbT, axis=0, keepdims=True))
    rowmax = jnp.concatenate(parts, axis=0)

    rowid = (jax.lax.broadcasted_iota(jnp.int32, (_NBLK, _BW), 0) * _BW
             + jax.lax.broadcasted_iota(jnp.int32, (_NBLK, _BW), 1))
    lane64 = jax.lax.broadcasted_iota(jnp.int32, (1, _TAKE), 1)
    sub64 = jax.lax.broadcasted_iota(jnp.int32, (_TAKE, 1), 0)
    col4096 = jax.lax.broadcasted_iota(jnp.int32, (1, _N), 1)

    bits = jax.lax.bitcast_convert_type(jax.nn.sigmoid(rowmax), jnp.int32)
    msb = jnp.int32(-(2 ** 31))
    srm = jnp.where(bits >= 0, bits, jnp.bitwise_xor(~bits, msb))
    t_u = jnp.zeros((), jnp.int32)
    for b in range(31, -1, -1):
        t_try = jnp.bitwise_or(t_u, jnp.int32(-(2 ** 31) if b == 31 else 1 << b))
        thr = jnp.bitwise_xor(t_try, msb)
        cnt = jnp.sum((srm >= thr).astype(jnp.int32))
        t_u = jnp.where(cnt >= _TAKE, t_try, t_u)
    t_s = jnp.bitwise_xor(t_u, msb)

    gt = srm > t_s
    eq = srm == t_s
    n_gt = jnp.sum(gt.astype(jnp.int32))
    need = _TAKE - n_gt

    li = jax.lax.broadcasted_iota(jnp.int32, (_BW, _BW), 0)
    lj = jax.lax.broadcasted_iota(jnp.int32, (_BW, _BW), 1)
    ut = (li <= lj).astype(jnp.float32)
    si = jax.lax.broadcasted_iota(jnp.int32, (_NBLK, _NBLK), 0)
    sj = jax.lax.broadcasted_iota(jnp.int32, (_NBLK, _NBLK), 1)
    lt = (sj < si).astype(jnp.float32)

    def excl_prefix(mask):
        mf = mask.astype(jnp.float32)
        incl = jax.lax.dot_general(mf, ut, (((1,), (0,)), ((), ())),
                                   preferred_element_type=jnp.float32)
        totals = jnp.sum(mf, axis=1, keepdims=True)
        offs = jax.lax.dot_general(lt, totals, (((1,), (0,)), ((), ())),
                                   preferred_element_type=jnp.float32)
        return (incl - mf + offs).astype(jnp.int32)

    eq_take = eq & (excl_prefix(eq) < need)
    sel = gt | eq_take
    slot = excl_prefix(sel)

    gath = jnp.zeros((_TAKE, _H), jnp.float32)
    ids = jnp.zeros((_TAKE, 1), jnp.float32)
    for blk in range(_NBLK):
        pb = (jnp.where(sel[blk:blk + 1, :], slot[blk:blk + 1, :], _IBIG)
              == sub64).astype(jnp.float32)
        gath = gath + jax.lax.dot_general(
            pb, subj[blk * _BW:(blk + 1) * _BW, :], (((1,), (0,)), ((), ())),
            preferred_element_type=jnp.float32)
        idc = jax.lax.broadcasted_iota(jnp.int32, (_BW, 1), 0) + blk * _BW
        idlo = (idc % 256).astype(jnp.float32)
        idhi = (idc // 256).astype(jnp.float32)
        ids = (ids
               + jax.lax.dot_general(pb, idlo, (((1,), (0,)), ((), ())),
                                     preferred_element_type=jnp.float32)
               + 256.0 * jax.lax.dot_general(pb, idhi, (((1,), (0,)), ((), ())),
                                             preferred_element_type=jnp.float32))
    sel_col = ids.astype(jnp.int32)

    L = jax.lax.dot_general(gath, obj, (((1,), (1,)), ((), ())),
                            preferred_element_type=jnp.float32)
    S = jax.nn.sigmoid(L)
    lrm = jnp.max(S, axis=1, keepdims=True)
    return S, sel_col, lrm


def _relpn_kernel(x_ref, W1s_ref, b1s_ref, W2s_ref, b2s_ref,
                  W1o_ref, b1o_ref, W2o_ref, b2o_ref,
                  pairs_ref, scores_ref, L0_ref, L1_ref):
    args = (W1s_ref[...], b1s_ref[...], W2s_ref[...], b2s_ref[...],
            W1o_ref[...], b1o_ref[...], W2o_ref[...], b2o_ref[...])
    S0, selc0, lrm0 = _prep_image(x_ref[0], *args)
    L0_ref[...] = S0
    S1, selc1, lrm1 = _prep_image(x_ref[1], *args)
    L1_ref[...] = S1

    lane64 = jax.lax.broadcasted_iota(jnp.int32, (1, _TAKE), 1)
    sub64 = jax.lax.broadcasted_iota(jnp.int32, (_TAKE, 1), 0)
    col4096 = jax.lax.broadcasted_iota(jnp.int32, (1, _N), 1)

    def pop1(L_ref, sel_col, t, lrm, scores, rvec, cvec):
        m = jnp.max(lrm)
        ro = jnp.min(jnp.where(lrm == m, sel_col, _IBIG))
        tloc = jnp.min(jnp.where((lrm == m) & (sel_col == ro), sub64, _IBIG))
        lrow = L_ref[pl.ds(tloc, 1), :]
        c = jnp.min(jnp.where(lrow == m, col4096, _IBIG))
        lrow2 = jnp.where(col4096 == c, _NEG, lrow)
        L_ref[pl.ds(tloc, 1), :] = lrow2
        lrm = jnp.where(sub64 == tloc, jnp.max(lrow2), lrm)
        scores = jnp.where(lane64 == t, m, scores)
        rvec = jnp.where(lane64 == t, ro, rvec)
        cvec = jnp.where(lane64 == t, c, cvec)
        return lrm, scores, rvec, cvec

    def pop(t, carry):
        st0, st1 = carry
        st0 = pop1(L0_ref, selc0, t, *st0)
        st1 = pop1(L1_ref, selc1, t, *st1)
        return st0, st1

    init = lambda lrm: (lrm,
                        jnp.full((1, _TAKE), _NEG, jnp.float32),
                        jnp.zeros((1, _TAKE), jnp.int32),
                        jnp.zeros((1, _TAKE), jnp.int32))
    (st0, st1) = jax.lax.fori_loop(0, _TAKE, pop, (init(lrm0), init(lrm1)))

    for b, (_, scores, rvec, cvec) in enumerate((st0, st1)):
        pairs_ref[b, 0:1, :] = rvec
        pairs_ref[b, 1:2, :] = cvec
        scores_ref[b, 0:1, :] = scores


@jax.jit
def kernel(class_logits, proposals, W1s, b1s, W2s, b2s, W1o, b1o, W2o, b2o):
    del proposals
    B = class_logits.shape[0]
    b1s2 = b1s.reshape(1, _H)
    b2s2 = b2s.reshape(1, _H)
    b1o2 = b1o.reshape(1, _H)
    b2o2 = b2o.reshape(1, _H)

    assert B == 2
    pairs2, scores = pl.pallas_call(
        _relpn_kernel,
        out_shape=[
            jax.ShapeDtypeStruct((B, 2, _TAKE), jnp.int32),
            jax.ShapeDtypeStruct((B, 1, _TAKE), jnp.float32),
        ],
        scratch_shapes=[
            pltpu.VMEM((_TAKE, _N), jnp.float32),
            pltpu.VMEM((_TAKE, _N), jnp.float32),
        ],
    )(class_logits, W1s, b1s2, W2s, b2s2, W1o, b1o2, W2o, b2o2)

    pairs = jnp.swapaxes(pairs2, 1, 2)
    return pairs, scores[:, 0, :]
